# baseline (speedup 1.0000x reference)
import math

import jax
import jax.numpy as jnp
from jax.experimental import pallas as pl
from jax.experimental.pallas import tpu as pltpu

EPS = 1e-6
THETA = 10000.0
NEG = -1e9
HD = 128
f32 = jnp.float32
bf16 = jnp.bfloat16


def _split(a):
    hi = a.astype(bf16)
    lo = (a - hi.astype(f32)).astype(bf16)
    return hi, lo


def _dot3(a, bh, bl, dims):
    dn = ((dims), ((), ()))
    ah, al = _split(a)
    r = jax.lax.dot_general(ah, bh, dn, preferred_element_type=f32)
    r += jax.lax.dot_general(ah, bl, dn, preferred_element_type=f32)
    r += jax.lax.dot_general(al, bh, dn, preferred_element_type=f32)
    return r


def _dot3f(a, b, dims):
    bh, bl = _split(b)
    return _dot3(a, bh, bl, dims)


def _router_kernel(hs_ref, w_ref, out_ref):
    hb = hs_ref[0].astype(bf16).astype(f32)
    wb = w_ref[...].astype(bf16).astype(f32)
    out_ref[0, 0, :] = jnp.sum(hb * wb, axis=1)


def _gather_kernel(idx_ref, hs_ref, sel_ref):
    b = pl.program_id(0)
    kk = sel_ref.shape[0]

    def body(j, carry):
        r = idx_ref[b, j]
        sel_ref[pl.ds(j, 1), :] = hs_ref[pl.ds(r, 1), :]
        return carry

    jax.lax.fori_loop(0, kk, body, 0)


def _rms(x, w):
    ms = jnp.mean(x * x, axis=1, keepdims=True)
    return x * jax.lax.rsqrt(ms + EPS) * w


def _proj_kernel(x_ref, wh_ref, wl_ref, b_ref, ln_ref, o_ref):
    h = _rms(x_ref[...], ln_ref[...])
    o_ref[...] = _dot3(h, wh_ref[...], wl_ref[...], ((1,), (0,))) + b_ref[...]


def _attn_kernel(pos_ref, q_ref, k_ref, v_ref, o_ref):
    pos = pos_ref[0]
    half = HD // 2
    i64 = jax.lax.broadcasted_iota(jnp.int32, (1, half), 1).astype(f32)
    inv = jnp.exp(i64 * (-math.log(THETA) / half))
    ang = pos * inv
    c = jnp.cos(ang)
    s = jnp.sin(ang)
    cos = jnp.concatenate([c, c], axis=1)
    sin = jnp.concatenate([s, s], axis=1)

    def rope(x):
        rh = jnp.concatenate([-x[:, half:], x[:, :half]], axis=1)
        return x * cos + rh * sin

    q = rope(q_ref[...])
    k = rope(k_ref[...])
    sc = _dot3f(q, k, ((1,), (1,))) * (1.0 / math.sqrt(HD))
    kk = sc.shape[0]
    ri = jax.lax.broadcasted_iota(jnp.int32, (kk, kk), 0)
    ci = jax.lax.broadcasted_iota(jnp.int32, (kk, kk), 1)
    sc = jnp.where(ri >= ci, sc, NEG)
    m = jnp.max(sc, axis=1, keepdims=True)
    e = jnp.exp(sc - m)
    p = e / jnp.sum(e, axis=1, keepdims=True)
    o_ref[...] = _dot3f(p, v_ref[...], ((1,), (0,)))


def _o_kernel(ctx_ref, wh_ref, wl_ref, sel_ref, x_ref):
    x_ref[...] = sel_ref[...] + _dot3(ctx_ref[...], wh_ref[...], wl_ref[...],
                                      ((1,), (0,)))


def _ff_kernel(x_ref, wh_ref, wl_ref, ln_ref, o1_ref, o2_ref):
    h = _rms(x_ref[...], ln_ref[...])
    r = _dot3(h, wh_ref[...], wl_ref[...], ((1,), (0,)))
    i1 = o1_ref.shape[1]
    o1_ref[...] = r[:, :i1]
    o2_ref[...] = r[:, i1:]


def _down1_kernel(g_ref, u_ref, x_ref, wh_ref, wl_ref, y_ref):
    g = g_ref[...]
    a = g * jax.nn.sigmoid(g) * u_ref[...]
    y_ref[...] = x_ref[...] + _dot3(a, wh_ref[...], wl_ref[...], ((1,), (0,)))


def _down2_kernel(g_ref, u_ref, y1_ref, ws_ref, wh_ref, wl_ref, y_ref):
    g = g_ref[...]
    a = g * jax.nn.sigmoid(g) * u_ref[...]
    y = y1_ref[...] + _dot3(a, wh_ref[...], wl_ref[...], ((1,), (0,)))
    y_ref[...] = y * ws_ref[...]


def _scatter_kernel(idx_ref, st_ref, y_ref, hs_ref, out_ref):
    b = pl.program_id(0)
    t = pl.program_id(1)
    st = out_ref.shape[0]
    out_ref[...] = hs_ref[...]
    lo = st_ref[b, t]
    hi = st_ref[b, t + 1]

    def body(j, carry):
        r = idx_ref[b, j] - t * st
        out_ref[pl.ds(r, 1), :] = y_ref[pl.ds(j, 1), :]
        return carry

    jax.lax.fori_loop(lo, hi, body, 0)


def kernel(hidden_states, position_ids, router_w, router_b, q_w, q_b, k_w,
           k_b, v_w, v_b, o_w, gate_w, up_w, down_w, ln1_w, ln2_w):
    B, S, D = hidden_states.shape
    INTER = gate_w.shape[0]
    H = D // HD
    kk = max(1, S // 2)
    BT = B * kk
    TT = min(256, BT)
    FT = min(128, BT)

    hs2 = hidden_states.reshape(B * S, D)

    q_wh, q_wl = _split(q_w.T)
    k_wh, k_wl = _split(k_w.T)
    v_wh, v_wl = _split(v_w.T)
    o_wh, o_wl = _split(o_w.T)
    g_wh, g_wl = _split(gate_w.T)
    u_wh, u_wl = _split(up_w.T)
    d_wt = down_w.T
    i1 = (INTER // 2 + 127) // 128 * 128
    i2 = INTER - i1
    d1h, d1l = _split(d_wt[:i1])
    d2h, d2l = _split(d_wt[i1:])

    vlim = pltpu.CompilerParams(vmem_limit_bytes=63 * 1024 * 1024)

    rw = pl.pallas_call(
        _router_kernel,
        grid=(B,),
        in_specs=[
            pl.BlockSpec((1, S, D), lambda b: (b, 0, 0)),
            pl.BlockSpec((1, D), lambda b: (0, 0)),
        ],
        out_specs=pl.BlockSpec((1, 1, S), lambda b: (b, 0, 0)),
        out_shape=jax.ShapeDtypeStruct((B, 1, S), f32),
    )(hidden_states, router_w)
    rw = rw.reshape(B, S) + router_b[0]

    _, top_idx = jax.lax.top_k(rw, kk)
    idx = jnp.sort(top_idx, axis=1).astype(jnp.int32)
    w_sel = jnp.take_along_axis(rw, idx, axis=1).reshape(BT, 1)
    pos_sel = jnp.take_along_axis(position_ids, idx, axis=1)
    pos_f = pos_sel.astype(f32).reshape(B, kk, 1)

    sel = pl.pallas_call(
        _gather_kernel,
        grid_spec=pltpu.PrefetchScalarGridSpec(
            num_scalar_prefetch=1,
            grid=(B,),
            in_specs=[pl.BlockSpec((S, D), lambda b, i_ref: (b, 0))],
            out_specs=pl.BlockSpec((kk, D), lambda b, i_ref: (b, 0)),
        ),
        out_shape=jax.ShapeDtypeStruct((BT, D), f32),
    )(idx, hs2)

    def proj(x, wh, wl, bias):
        return pl.pallas_call(
            _proj_kernel,
            grid=(BT // TT,),
            in_specs=[
                pl.BlockSpec((TT, D), lambda t: (t, 0)),
                pl.BlockSpec((D, D), lambda t: (0, 0)),
                pl.BlockSpec((D, D), lambda t: (0, 0)),
                pl.BlockSpec((D,), lambda t: (0,)),
                pl.BlockSpec((D,), lambda t: (0,)),
            ],
            out_specs=pl.BlockSpec((TT, D), lambda t: (t, 0)),
            out_shape=jax.ShapeDtypeStruct((BT, D), f32),
            compiler_params=vlim,
        )(x, wh, wl, bias, ln1_w)

    q = proj(sel, q_wh, q_wl, q_b)
    k = proj(sel, k_wh, k_wl, k_b)
    v = proj(sel, v_wh, v_wl, v_b)

    ctx = pl.pallas_call(
        _attn_kernel,
        grid=(B, H),
        in_specs=[
            pl.BlockSpec((1, kk, 1), lambda b, h: (b, 0, 0)),
            pl.BlockSpec((kk, HD), lambda b, h: (b, h)),
            pl.BlockSpec((kk, HD), lambda b, h: (b, h)),
            pl.BlockSpec((kk, HD), lambda b, h: (b, h)),
        ],
        out_specs=pl.BlockSpec((kk, HD), lambda b, h: (b, h)),
        out_shape=jax.ShapeDtypeStruct((BT, D), f32),
        compiler_params=vlim,
    )(pos_f, q, k, v)

    x = pl.pallas_call(
        _o_kernel,
        grid=(BT // TT,),
        in_specs=[
            pl.BlockSpec((TT, D), lambda t: (t, 0)),
            pl.BlockSpec((D, D), lambda t: (0, 0)),
            pl.BlockSpec((D, D), lambda t: (0, 0)),
            pl.BlockSpec((TT, D), lambda t: (t, 0)),
        ],
        out_specs=pl.BlockSpec((TT, D), lambda t: (t, 0)),
        out_shape=jax.ShapeDtypeStruct((BT, D), f32),
        compiler_params=vlim,
    )(ctx, o_wh, o_wl, sel)

    def ff(xin, wh, wl):
        return pl.pallas_call(
            _ff_kernel,
            grid=(BT // FT,),
            in_specs=[
                pl.BlockSpec((FT, D), lambda t: (t, 0)),
                pl.BlockSpec((D, INTER), lambda t: (0, 0)),
                pl.BlockSpec((D, INTER), lambda t: (0, 0)),
                pl.BlockSpec((D,), lambda t: (0,)),
            ],
            out_specs=[
                pl.BlockSpec((FT, i1), lambda t: (t, 0)),
                pl.BlockSpec((FT, i2), lambda t: (t, 0)),
            ],
            out_shape=[
                jax.ShapeDtypeStruct((BT, i1), f32),
                jax.ShapeDtypeStruct((BT, i2), f32),
            ],
            compiler_params=vlim,
        )(xin, wh, wl, ln2_w)

    g1, g2 = ff(x, g_wh, g_wl)
    u1, u2 = ff(x, u_wh, u_wl)

    y1 = pl.pallas_call(
        _down1_kernel,
        grid=(BT // FT,),
        in_specs=[
            pl.BlockSpec((FT, i1), lambda t: (t, 0)),
            pl.BlockSpec((FT, i1), lambda t: (t, 0)),
            pl.BlockSpec((FT, D), lambda t: (t, 0)),
            pl.BlockSpec((i1, D), lambda t: (0, 0)),
            pl.BlockSpec((i1, D), lambda t: (0, 0)),
        ],
        out_specs=pl.BlockSpec((FT, D), lambda t: (t, 0)),
        out_shape=jax.ShapeDtypeStruct((BT, D), f32),
        compiler_params=vlim,
    )(g1, u1, x, d1h, d1l)

    y = pl.pallas_call(
        _down2_kernel,
        grid=(BT // FT,),
        in_specs=[
            pl.BlockSpec((FT, i2), lambda t: (t, 0)),
            pl.BlockSpec((FT, i2), lambda t: (t, 0)),
            pl.BlockSpec((FT, D), lambda t: (t, 0)),
            pl.BlockSpec((FT, 1), lambda t: (t, 0)),
            pl.BlockSpec((i2, D), lambda t: (0, 0)),
            pl.BlockSpec((i2, D), lambda t: (0, 0)),
        ],
        out_specs=pl.BlockSpec((FT, D), lambda t: (t, 0)),
        out_shape=jax.ShapeDtypeStruct((BT, D), f32),
        compiler_params=vlim,
    )(g2, u2, y1, w_sel, d2h, d2l)

    ST = min(1024, S)
    n_st = S // ST
    bases = jnp.arange(0, S + 1, ST, dtype=jnp.int32)[None, :]
    starts = jnp.sum(idx[:, :, None] < bases[:, None, :], axis=1).astype(
        jnp.int32)

    out = pl.pallas_call(
        _scatter_kernel,
        grid_spec=pltpu.PrefetchScalarGridSpec(
            num_scalar_prefetch=2,
            grid=(B, n_st),
            in_specs=[
                pl.BlockSpec((kk, D), lambda b, t, i_ref, s_ref: (b, 0)),
                pl.BlockSpec((ST, D), lambda b, t, i_ref, s_ref: (b * n_st + t, 0)),
            ],
            out_specs=pl.BlockSpec((ST, D), lambda b, t, i_ref, s_ref: (b * n_st + t, 0)),
        ),
        out_shape=jax.ShapeDtypeStruct((B * S, D), f32),
    )(idx, starts, y, hs2)

    return out.reshape(B, S, D)


# speedup vs baseline: 2.0315x; 2.0315x over previous
import math

import jax
import jax.numpy as jnp
from jax.experimental import pallas as pl
from jax.experimental.pallas import tpu as pltpu

EPS = 1e-6
THETA = 10000.0
NEG = -1e9
HD = 128
f32 = jnp.float32
bf16 = jnp.bfloat16


def _dot1(a, b, dims):
    dn = ((dims), ((), ()))
    return jax.lax.dot_general(a.astype(bf16), b, dn,
                               preferred_element_type=f32)


def _dot1f(a, b, dims):
    return _dot1(a, b.astype(bf16), dims)


def _router_kernel(hs_ref, w_ref, out_ref):
    hb = hs_ref[0].astype(bf16).astype(f32)
    wb = w_ref[...].astype(bf16).astype(f32)
    out_ref[0, 0, :] = jnp.sum(hb * wb, axis=1)


def _gather_kernel(idx_ref, hs_ref, sel_ref):
    b = pl.program_id(0)
    kk = sel_ref.shape[0]

    def body(j, carry):
        r = idx_ref[b, j]
        sel_ref[pl.ds(j, 1), :] = hs_ref[pl.ds(r, 1), :]
        return carry

    jax.lax.fori_loop(0, kk, body, 0)


def _rms(x, w):
    ms = jnp.mean(x * x, axis=1, keepdims=True)
    return x * jax.lax.rsqrt(ms + EPS) * w


def _proj_kernel(x_ref, w_ref, b_ref, ln_ref, o_ref):
    h = _rms(x_ref[...], ln_ref[...])
    o_ref[...] = _dot1(h, w_ref[...], ((1,), (0,))) + b_ref[...]


def _attn_kernel(pos_ref, q_ref, k_ref, v_ref, o_ref):
    pos = pos_ref[0]
    half = HD // 2
    i64 = jax.lax.broadcasted_iota(jnp.int32, (1, half), 1).astype(f32)
    inv = jnp.exp(i64 * (-math.log(THETA) / half))
    ang = pos * inv
    c = jnp.cos(ang)
    s = jnp.sin(ang)
    cos = jnp.concatenate([c, c], axis=1)
    sin = jnp.concatenate([s, s], axis=1)

    def rope(x):
        rh = jnp.concatenate([-x[:, half:], x[:, :half]], axis=1)
        return x * cos + rh * sin

    q = rope(q_ref[...])
    k = rope(k_ref[...])
    sc = _dot1f(q, k, ((1,), (1,))) * (1.0 / math.sqrt(HD))
    kk = sc.shape[0]
    ri = jax.lax.broadcasted_iota(jnp.int32, (kk, kk), 0)
    ci = jax.lax.broadcasted_iota(jnp.int32, (kk, kk), 1)
    sc = jnp.where(ri >= ci, sc, NEG)
    m = jnp.max(sc, axis=1, keepdims=True)
    e = jnp.exp(sc - m)
    p = e / jnp.sum(e, axis=1, keepdims=True)
    o_ref[...] = _dot1f(p, v_ref[...], ((1,), (0,)))


def _o_kernel(ctx_ref, w_ref, sel_ref, x_ref):
    x_ref[...] = sel_ref[...] + _dot1(ctx_ref[...], w_ref[...], ((1,), (0,)))


def _mlp1_kernel(x_ref, gw_ref, uw_ref, ln_ref, a_ref):
    h = _rms(x_ref[...], ln_ref[...])
    g = _dot1(h, gw_ref[...], ((1,), (0,)))
    u = _dot1(h, uw_ref[...], ((1,), (0,)))
    a_ref[...] = g * jax.nn.sigmoid(g) * u


def _down_kernel(a_ref, x_ref, ws_ref, w_ref, y_ref):
    y = x_ref[...] + _dot1(a_ref[...], w_ref[...], ((1,), (0,)))
    y_ref[...] = y * ws_ref[...]


def _scatter_kernel(idx_ref, st_ref, y_ref, hs_ref, out_ref):
    b = pl.program_id(0)
    t = pl.program_id(1)
    st = out_ref.shape[0]
    out_ref[...] = hs_ref[...]
    lo = st_ref[b, t]
    hi = st_ref[b, t + 1]

    def body(j, carry):
        r = idx_ref[b, j] - t * st
        out_ref[pl.ds(r, 1), :] = y_ref[pl.ds(j, 1), :]
        return carry

    jax.lax.fori_loop(lo, hi, body, 0)


def kernel(hidden_states, position_ids, router_w, router_b, q_w, q_b, k_w,
           k_b, v_w, v_b, o_w, gate_w, up_w, down_w, ln1_w, ln2_w):
    B, S, D = hidden_states.shape
    INTER = gate_w.shape[0]
    H = D // HD
    kk = max(1, S // 2)
    BT = B * kk
    TT = min(256, BT)
    FT = min(128, BT)

    hs2 = hidden_states.reshape(B * S, D)

    q_wt = q_w.T.astype(bf16)
    k_wt = k_w.T.astype(bf16)
    v_wt = v_w.T.astype(bf16)
    o_wt = o_w.T.astype(bf16)
    g_wt = gate_w.T.astype(bf16)
    u_wt = up_w.T.astype(bf16)
    d_wt = down_w.T.astype(bf16)

    vlim = pltpu.CompilerParams(vmem_limit_bytes=63 * 1024 * 1024)

    rw = pl.pallas_call(
        _router_kernel,
        grid=(B,),
        in_specs=[
            pl.BlockSpec((1, S, D), lambda b: (b, 0, 0)),
            pl.BlockSpec((1, D), lambda b: (0, 0)),
        ],
        out_specs=pl.BlockSpec((1, 1, S), lambda b: (b, 0, 0)),
        out_shape=jax.ShapeDtypeStruct((B, 1, S), f32),
    )(hidden_states, router_w)
    rw = rw.reshape(B, S) + router_b[0]

    _, top_idx = jax.lax.top_k(rw, kk)
    idx = jnp.sort(top_idx, axis=1).astype(jnp.int32)
    w_sel = jnp.take_along_axis(rw, idx, axis=1).reshape(BT, 1)
    pos_sel = jnp.take_along_axis(position_ids, idx, axis=1)
    pos_f = pos_sel.astype(f32).reshape(B, kk, 1)

    sel = pl.pallas_call(
        _gather_kernel,
        grid_spec=pltpu.PrefetchScalarGridSpec(
            num_scalar_prefetch=1,
            grid=(B,),
            in_specs=[pl.BlockSpec((S, D), lambda b, i_ref: (b, 0))],
            out_specs=pl.BlockSpec((kk, D), lambda b, i_ref: (b, 0)),
        ),
        out_shape=jax.ShapeDtypeStruct((BT, D), f32),
    )(idx, hs2)

    def proj(x, wt, bias):
        return pl.pallas_call(
            _proj_kernel,
            grid=(BT // TT,),
            in_specs=[
                pl.BlockSpec((TT, D), lambda t: (t, 0)),
                pl.BlockSpec((D, D), lambda t: (0, 0)),
                pl.BlockSpec((D,), lambda t: (0,)),
                pl.BlockSpec((D,), lambda t: (0,)),
            ],
            out_specs=pl.BlockSpec((TT, D), lambda t: (t, 0)),
            out_shape=jax.ShapeDtypeStruct((BT, D), f32),
            compiler_params=vlim,
        )(x, wt, bias, ln1_w)

    q = proj(sel, q_wt, q_b)
    k = proj(sel, k_wt, k_b)
    v = proj(sel, v_wt, v_b)

    ctx = pl.pallas_call(
        _attn_kernel,
        grid=(B, H),
        in_specs=[
            pl.BlockSpec((1, kk, 1), lambda b, h: (b, 0, 0)),
            pl.BlockSpec((kk, HD), lambda b, h: (b, h)),
            pl.BlockSpec((kk, HD), lambda b, h: (b, h)),
            pl.BlockSpec((kk, HD), lambda b, h: (b, h)),
        ],
        out_specs=pl.BlockSpec((kk, HD), lambda b, h: (b, h)),
        out_shape=jax.ShapeDtypeStruct((BT, D), f32),
        compiler_params=vlim,
    )(pos_f, q, k, v)

    x = pl.pallas_call(
        _o_kernel,
        grid=(BT // TT,),
        in_specs=[
            pl.BlockSpec((TT, D), lambda t: (t, 0)),
            pl.BlockSpec((D, D), lambda t: (0, 0)),
            pl.BlockSpec((TT, D), lambda t: (t, 0)),
        ],
        out_specs=pl.BlockSpec((TT, D), lambda t: (t, 0)),
        out_shape=jax.ShapeDtypeStruct((BT, D), f32),
        compiler_params=vlim,
    )(ctx, o_wt, sel)

    act = pl.pallas_call(
        _mlp1_kernel,
        grid=(BT // FT,),
        in_specs=[
            pl.BlockSpec((FT, D), lambda t: (t, 0)),
            pl.BlockSpec((D, INTER), lambda t: (0, 0)),
            pl.BlockSpec((D, INTER), lambda t: (0, 0)),
            pl.BlockSpec((D,), lambda t: (0,)),
        ],
        out_specs=pl.BlockSpec((FT, INTER), lambda t: (t, 0)),
        out_shape=jax.ShapeDtypeStruct((BT, INTER), f32),
        compiler_params=vlim,
    )(x, g_wt, u_wt, ln2_w)

    y = pl.pallas_call(
        _down_kernel,
        grid=(BT // FT,),
        in_specs=[
            pl.BlockSpec((FT, INTER), lambda t: (t, 0)),
            pl.BlockSpec((FT, D), lambda t: (t, 0)),
            pl.BlockSpec((FT, 1), lambda t: (t, 0)),
            pl.BlockSpec((INTER, D), lambda t: (0, 0)),
        ],
        out_specs=pl.BlockSpec((FT, D), lambda t: (t, 0)),
        out_shape=jax.ShapeDtypeStruct((BT, D), f32),
        compiler_params=vlim,
    )(act, x, w_sel, d_wt)

    ST = min(1024, S)
    n_st = S // ST
    bases = jnp.arange(0, S + 1, ST, dtype=jnp.int32)[None, :]
    starts = jnp.sum(idx[:, :, None] < bases[:, None, :], axis=1).astype(
        jnp.int32)

    out = pl.pallas_call(
        _scatter_kernel,
        grid_spec=pltpu.PrefetchScalarGridSpec(
            num_scalar_prefetch=2,
            grid=(B, n_st),
            in_specs=[
                pl.BlockSpec((kk, D), lambda b, t, i_ref, s_ref: (b, 0)),
                pl.BlockSpec((ST, D), lambda b, t, i_ref, s_ref: (b * n_st + t, 0)),
            ],
            out_specs=pl.BlockSpec((ST, D), lambda b, t, i_ref, s_ref: (b * n_st + t, 0)),
        ),
        out_shape=jax.ShapeDtypeStruct((B * S, D), f32),
    )(idx, starts, y, hs2)

    return out.reshape(B, S, D)


# speedup vs baseline: 2.0412x; 1.0047x over previous
import math

import jax
import jax.numpy as jnp
from jax.experimental import pallas as pl
from jax.experimental.pallas import tpu as pltpu

EPS = 1e-6
THETA = 10000.0
NEG = -1e9
HD = 128
f32 = jnp.float32
bf16 = jnp.bfloat16


def _dot1(a, b, dims):
    dn = ((dims), ((), ()))
    return jax.lax.dot_general(a.astype(bf16), b, dn,
                               preferred_element_type=f32)


def _dot1f(a, b, dims):
    return _dot1(a, b.astype(bf16), dims)


def _router_kernel(hs_ref, w_ref, out_ref):
    hb = hs_ref[0].astype(bf16).astype(f32)
    wb = w_ref[...].astype(bf16).astype(f32)
    out_ref[0, 0, :] = jnp.sum(hb * wb, axis=1)


def _gather_kernel(idx_ref, hs_ref, sel_ref):
    b = pl.program_id(0)
    kk = sel_ref.shape[0]

    def body(j, carry):
        r = idx_ref[b, j]
        sel_ref[pl.ds(j, 1), :] = hs_ref[pl.ds(r, 1), :]
        return carry

    jax.lax.fori_loop(0, kk, body, 0)


def _rms(x, w):
    ms = jnp.mean(x * x, axis=1, keepdims=True)
    return x * jax.lax.rsqrt(ms + EPS) * w


def _proj_kernel(x_ref, w_ref, b_ref, ln_ref, o_ref):
    h = _rms(x_ref[...], ln_ref[...])
    o_ref[...] = (_dot1(h, w_ref[...], ((1,), (0,))) +
                  b_ref[...]).astype(bf16)


def _attn_kernel(pos_ref, q_ref, k_ref, v_ref, o_ref):
    pos = pos_ref[0]
    half = HD // 2
    i64 = jax.lax.broadcasted_iota(jnp.int32, (1, half), 1).astype(f32)
    inv = jnp.exp(i64 * (-math.log(THETA) / half))
    ang = pos * inv
    c = jnp.cos(ang)
    s = jnp.sin(ang)
    cos = jnp.concatenate([c, c], axis=1)
    sin = jnp.concatenate([s, s], axis=1)

    def rope(x):
        rh = jnp.concatenate([-x[:, half:], x[:, :half]], axis=1)
        return x * cos + rh * sin

    q = rope(q_ref[...].astype(f32))
    k = rope(k_ref[...].astype(f32))
    sc = _dot1f(q, k, ((1,), (1,))) * (1.0 / math.sqrt(HD))
    kk = sc.shape[0]
    ri = jax.lax.broadcasted_iota(jnp.int32, (kk, kk), 0)
    ci = jax.lax.broadcasted_iota(jnp.int32, (kk, kk), 1)
    sc = jnp.where(ri >= ci, sc, NEG)
    m = jnp.max(sc, axis=1, keepdims=True)
    e = jnp.exp(sc - m)
    p = e / jnp.sum(e, axis=1, keepdims=True)
    o_ref[...] = _dot1(p, v_ref[...], ((1,), (0,))).astype(bf16)


def _o_kernel(ctx_ref, w_ref, sel_ref, x_ref):
    x_ref[...] = sel_ref[...] + _dot1(ctx_ref[...], w_ref[...], ((1,), (0,)))


def _mlp1_kernel(x_ref, gw_ref, uw_ref, ln_ref, a_ref):
    h = _rms(x_ref[...], ln_ref[...])
    g = _dot1(h, gw_ref[...], ((1,), (0,)))
    u = _dot1(h, uw_ref[...], ((1,), (0,)))
    a_ref[...] = (g * jax.nn.sigmoid(g) * u).astype(bf16)


def _down_kernel(a_ref, x_ref, ws_ref, w_ref, y_ref):
    y = x_ref[...] + _dot1(a_ref[...], w_ref[...], ((1,), (0,)))
    y_ref[...] = y * ws_ref[...]


def _scatter_kernel(idx_ref, st_ref, y_ref, hs_ref, out_ref):
    b = pl.program_id(0)
    t = pl.program_id(1)
    st = out_ref.shape[0]
    out_ref[...] = hs_ref[...]
    lo = st_ref[b, t]
    hi = st_ref[b, t + 1]

    def body(j, carry):
        r = idx_ref[b, j] - t * st
        out_ref[pl.ds(r, 1), :] = y_ref[pl.ds(j, 1), :]
        return carry

    jax.lax.fori_loop(lo, hi, body, 0)


def kernel(hidden_states, position_ids, router_w, router_b, q_w, q_b, k_w,
           k_b, v_w, v_b, o_w, gate_w, up_w, down_w, ln1_w, ln2_w):
    B, S, D = hidden_states.shape
    INTER = gate_w.shape[0]
    H = D // HD
    kk = max(1, S // 2)
    BT = B * kk
    TT = min(256, BT)
    FT = min(128, BT)

    hs2 = hidden_states.reshape(B * S, D)

    q_wt = q_w.T.astype(bf16)
    k_wt = k_w.T.astype(bf16)
    v_wt = v_w.T.astype(bf16)
    o_wt = o_w.T.astype(bf16)
    g_wt = gate_w.T.astype(bf16)
    u_wt = up_w.T.astype(bf16)
    d_wt = down_w.T.astype(bf16)

    vlim = pltpu.CompilerParams(vmem_limit_bytes=63 * 1024 * 1024)

    rw = pl.pallas_call(
        _router_kernel,
        grid=(B,),
        in_specs=[
            pl.BlockSpec((1, S, D), lambda b: (b, 0, 0)),
            pl.BlockSpec((1, D), lambda b: (0, 0)),
        ],
        out_specs=pl.BlockSpec((1, 1, S), lambda b: (b, 0, 0)),
        out_shape=jax.ShapeDtypeStruct((B, 1, S), f32),
    )(hidden_states, router_w)
    rw = rw.reshape(B, S) + router_b[0]

    _, top_idx = jax.lax.top_k(rw, kk)
    idx = jnp.sort(top_idx, axis=1).astype(jnp.int32)
    w_sel = jnp.take_along_axis(rw, idx, axis=1).reshape(BT, 1)
    pos_sel = jnp.take_along_axis(position_ids, idx, axis=1)
    pos_f = pos_sel.astype(f32).reshape(B, kk, 1)

    sel = pl.pallas_call(
        _gather_kernel,
        grid_spec=pltpu.PrefetchScalarGridSpec(
            num_scalar_prefetch=1,
            grid=(B,),
            in_specs=[pl.BlockSpec((S, D), lambda b, i_ref: (b, 0))],
            out_specs=pl.BlockSpec((kk, D), lambda b, i_ref: (b, 0)),
        ),
        out_shape=jax.ShapeDtypeStruct((BT, D), f32),
    )(idx, hs2)

    def proj(x, wt, bias):
        return pl.pallas_call(
            _proj_kernel,
            grid=(BT // TT,),
            in_specs=[
                pl.BlockSpec((TT, D), lambda t: (t, 0)),
                pl.BlockSpec((D, D), lambda t: (0, 0)),
                pl.BlockSpec((D,), lambda t: (0,)),
                pl.BlockSpec((D,), lambda t: (0,)),
            ],
            out_specs=pl.BlockSpec((TT, D), lambda t: (t, 0)),
            out_shape=jax.ShapeDtypeStruct((BT, D), bf16),
            compiler_params=vlim,
        )(x, wt, bias, ln1_w)

    q = proj(sel, q_wt, q_b)
    k = proj(sel, k_wt, k_b)
    v = proj(sel, v_wt, v_b)

    ctx = pl.pallas_call(
        _attn_kernel,
        grid=(B, H),
        in_specs=[
            pl.BlockSpec((1, kk, 1), lambda b, h: (b, 0, 0)),
            pl.BlockSpec((kk, HD), lambda b, h: (b, h)),
            pl.BlockSpec((kk, HD), lambda b, h: (b, h)),
            pl.BlockSpec((kk, HD), lambda b, h: (b, h)),
        ],
        out_specs=pl.BlockSpec((kk, HD), lambda b, h: (b, h)),
        out_shape=jax.ShapeDtypeStruct((BT, D), bf16),
        compiler_params=vlim,
    )(pos_f, q, k, v)

    x = pl.pallas_call(
        _o_kernel,
        grid=(BT // TT,),
        in_specs=[
            pl.BlockSpec((TT, D), lambda t: (t, 0)),
            pl.BlockSpec((D, D), lambda t: (0, 0)),
            pl.BlockSpec((TT, D), lambda t: (t, 0)),
        ],
        out_specs=pl.BlockSpec((TT, D), lambda t: (t, 0)),
        out_shape=jax.ShapeDtypeStruct((BT, D), f32),
        compiler_params=vlim,
    )(ctx, o_wt, sel)

    act = pl.pallas_call(
        _mlp1_kernel,
        grid=(BT // FT,),
        in_specs=[
            pl.BlockSpec((FT, D), lambda t: (t, 0)),
            pl.BlockSpec((D, INTER), lambda t: (0, 0)),
            pl.BlockSpec((D, INTER), lambda t: (0, 0)),
            pl.BlockSpec((D,), lambda t: (0,)),
        ],
        out_specs=pl.BlockSpec((FT, INTER), lambda t: (t, 0)),
        out_shape=jax.ShapeDtypeStruct((BT, INTER), bf16),
        compiler_params=vlim,
    )(x, g_wt, u_wt, ln2_w)

    y = pl.pallas_call(
        _down_kernel,
        grid=(BT // FT,),
        in_specs=[
            pl.BlockSpec((FT, INTER), lambda t: (t, 0)),
            pl.BlockSpec((FT, D), lambda t: (t, 0)),
            pl.BlockSpec((FT, 1), lambda t: (t, 0)),
            pl.BlockSpec((INTER, D), lambda t: (0, 0)),
        ],
        out_specs=pl.BlockSpec((FT, D), lambda t: (t, 0)),
        out_shape=jax.ShapeDtypeStruct((BT, D), f32),
        compiler_params=vlim,
    )(act, x, w_sel, d_wt)

    ST = min(1024, S)
    n_st = S // ST
    bases = jnp.arange(0, S + 1, ST, dtype=jnp.int32)[None, :]
    starts = jnp.sum(idx[:, :, None] < bases[:, None, :], axis=1).astype(
        jnp.int32)

    out = pl.pallas_call(
        _scatter_kernel,
        grid_spec=pltpu.PrefetchScalarGridSpec(
            num_scalar_prefetch=2,
            grid=(B, n_st),
            in_specs=[
                pl.BlockSpec((kk, D), lambda b, t, i_ref, s_ref: (b, 0)),
                pl.BlockSpec((ST, D), lambda b, t, i_ref, s_ref: (b * n_st + t, 0)),
            ],
            out_specs=pl.BlockSpec((ST, D), lambda b, t, i_ref, s_ref: (b * n_st + t, 0)),
        ),
        out_shape=jax.ShapeDtypeStruct((B * S, D), f32),
    )(idx, starts, y, hs2)

    return out.reshape(B, S, D)


# speedup vs baseline: 2.0906x; 1.0242x over previous
import math

import jax
import jax.numpy as jnp
from jax.experimental import pallas as pl
from jax.experimental.pallas import tpu as pltpu

EPS = 1e-6
THETA = 10000.0
NEG = -1e9
HD = 128
f32 = jnp.float32
bf16 = jnp.bfloat16


def _dot1(a, b, dims):
    dn = ((dims), ((), ()))
    return jax.lax.dot_general(a.astype(bf16), b, dn,
                               preferred_element_type=f32)


def _dot1f(a, b, dims):
    return _dot1(a, b.astype(bf16), dims)


def _router_kernel(hs_ref, w_ref, out_ref):
    hb = hs_ref[0].astype(bf16).astype(f32)
    wb = w_ref[...].astype(bf16).astype(f32)
    out_ref[0, 0, :] = jnp.sum(hb * wb, axis=1)


def _gather_kernel(idx_ref, hs_ref, sel_ref):
    b = pl.program_id(0)
    kk = sel_ref.shape[0]

    def body(j, carry):
        r = idx_ref[b, j]
        sel_ref[pl.ds(j, 1), :] = hs_ref[pl.ds(r, 1), :]
        return carry

    jax.lax.fori_loop(0, kk, body, 0)


def _rms(x, w):
    ms = jnp.mean(x * x, axis=1, keepdims=True)
    return x * jax.lax.rsqrt(ms + EPS) * w


def _qkv_kernel(x_ref, qw_ref, kw_ref, vw_ref, qb_ref, kb_ref, vb_ref,
                ln_ref, q_ref, k_ref, v_ref):
    h = _rms(x_ref[...], ln_ref[...])
    hb = h.astype(bf16)
    dn = (((1,), (0,)), ((), ()))
    q_ref[...] = (jax.lax.dot_general(hb, qw_ref[...], dn,
                  preferred_element_type=f32) + qb_ref[...]).astype(bf16)
    k_ref[...] = (jax.lax.dot_general(hb, kw_ref[...], dn,
                  preferred_element_type=f32) + kb_ref[...]).astype(bf16)
    v_ref[...] = (jax.lax.dot_general(hb, vw_ref[...], dn,
                  preferred_element_type=f32) + vb_ref[...]).astype(bf16)


def _attn_kernel(pos_ref, q_ref, k_ref, v_ref, o_ref):
    pos = pos_ref[0]
    half = HD // 2
    i64 = jax.lax.broadcasted_iota(jnp.int32, (1, half), 1).astype(f32)
    inv = jnp.exp(i64 * (-math.log(THETA) / half))
    ang = pos * inv
    c = jnp.cos(ang)
    s = jnp.sin(ang)
    cos = jnp.concatenate([c, c], axis=1)
    sin = jnp.concatenate([s, s], axis=1)

    def rope(x):
        rh = jnp.concatenate([-x[:, half:], x[:, :half]], axis=1)
        return x * cos + rh * sin

    q = rope(q_ref[...].astype(f32))
    k = rope(k_ref[...].astype(f32))
    sc = _dot1f(q, k, ((1,), (1,))) * (1.0 / math.sqrt(HD))
    kk = sc.shape[0]
    ri = jax.lax.broadcasted_iota(jnp.int32, (kk, kk), 0)
    ci = jax.lax.broadcasted_iota(jnp.int32, (kk, kk), 1)
    sc = jnp.where(ri >= ci, sc, NEG)
    m = jnp.max(sc, axis=1, keepdims=True)
    e = jnp.exp(sc - m)
    p = e / jnp.sum(e, axis=1, keepdims=True)
    o_ref[...] = _dot1(p, v_ref[...], ((1,), (0,))).astype(bf16)


def _o_kernel(ctx_ref, w_ref, sel_ref, x_ref):
    x_ref[...] = sel_ref[...] + _dot1(ctx_ref[...], w_ref[...], ((1,), (0,)))


def _mlp1_kernel(x_ref, gw_ref, uw_ref, ln_ref, a_ref):
    h = _rms(x_ref[...], ln_ref[...])
    g = _dot1(h, gw_ref[...], ((1,), (0,)))
    u = _dot1(h, uw_ref[...], ((1,), (0,)))
    a_ref[...] = (g * jax.nn.sigmoid(g) * u).astype(bf16)


def _down_kernel(a_ref, x_ref, ws_ref, w_ref, y_ref):
    y = x_ref[...] + _dot1(a_ref[...], w_ref[...], ((1,), (0,)))
    y_ref[...] = y * ws_ref[...]


def _scatter_kernel(idx_ref, st_ref, y_ref, hs_ref, out_ref):
    b = pl.program_id(0)
    t = pl.program_id(1)
    st = out_ref.shape[0]
    out_ref[...] = hs_ref[...]
    lo = st_ref[b, t]
    hi = st_ref[b, t + 1]

    def body(j, carry):
        r = idx_ref[b, j] - t * st
        out_ref[pl.ds(r, 1), :] = y_ref[pl.ds(j, 1), :]
        return carry

    jax.lax.fori_loop(lo, hi, body, 0)


def kernel(hidden_states, position_ids, router_w, router_b, q_w, q_b, k_w,
           k_b, v_w, v_b, o_w, gate_w, up_w, down_w, ln1_w, ln2_w):
    B, S, D = hidden_states.shape
    INTER = gate_w.shape[0]
    H = D // HD
    kk = max(1, S // 2)
    BT = B * kk
    TT = min(256, BT)
    FT = min(256, BT)

    hs2 = hidden_states.reshape(B * S, D)

    q_wt = q_w.T.astype(bf16)
    k_wt = k_w.T.astype(bf16)
    v_wt = v_w.T.astype(bf16)
    o_wt = o_w.T.astype(bf16)
    g_wt = gate_w.T.astype(bf16)
    u_wt = up_w.T.astype(bf16)
    d_wt = down_w.T.astype(bf16)

    vlim = pltpu.CompilerParams(vmem_limit_bytes=63 * 1024 * 1024)

    rw = pl.pallas_call(
        _router_kernel,
        grid=(B,),
        in_specs=[
            pl.BlockSpec((1, S, D), lambda b: (b, 0, 0)),
            pl.BlockSpec((1, D), lambda b: (0, 0)),
        ],
        out_specs=pl.BlockSpec((1, 1, S), lambda b: (b, 0, 0)),
        out_shape=jax.ShapeDtypeStruct((B, 1, S), f32),
    )(hidden_states, router_w)
    rw = rw.reshape(B, S) + router_b[0]

    _, top_idx = jax.lax.top_k(rw, kk)
    idx = jnp.sort(top_idx, axis=1).astype(jnp.int32)
    w_sel = jnp.take_along_axis(rw, idx, axis=1).reshape(BT, 1)
    pos_sel = jnp.take_along_axis(position_ids, idx, axis=1)
    pos_f = pos_sel.astype(f32).reshape(B, kk, 1)

    sel = pl.pallas_call(
        _gather_kernel,
        grid_spec=pltpu.PrefetchScalarGridSpec(
            num_scalar_prefetch=1,
            grid=(B,),
            in_specs=[pl.BlockSpec((S, D), lambda b, i_ref: (b, 0))],
            out_specs=pl.BlockSpec((kk, D), lambda b, i_ref: (b, 0)),
        ),
        out_shape=jax.ShapeDtypeStruct((BT, D), f32),
    )(idx, hs2)

    q, k, v = pl.pallas_call(
        _qkv_kernel,
        grid=(BT // TT,),
        in_specs=[
            pl.BlockSpec((TT, D), lambda t: (t, 0)),
            pl.BlockSpec((D, D), lambda t: (0, 0)),
            pl.BlockSpec((D, D), lambda t: (0, 0)),
            pl.BlockSpec((D, D), lambda t: (0, 0)),
            pl.BlockSpec((D,), lambda t: (0,)),
            pl.BlockSpec((D,), lambda t: (0,)),
            pl.BlockSpec((D,), lambda t: (0,)),
            pl.BlockSpec((D,), lambda t: (0,)),
        ],
        out_specs=[
            pl.BlockSpec((TT, D), lambda t: (t, 0)),
            pl.BlockSpec((TT, D), lambda t: (t, 0)),
            pl.BlockSpec((TT, D), lambda t: (t, 0)),
        ],
        out_shape=[
            jax.ShapeDtypeStruct((BT, D), bf16),
            jax.ShapeDtypeStruct((BT, D), bf16),
            jax.ShapeDtypeStruct((BT, D), bf16),
        ],
        compiler_params=vlim,
    )(sel, q_wt, k_wt, v_wt, q_b, k_b, v_b, ln1_w)

    ctx = pl.pallas_call(
        _attn_kernel,
        grid=(B, H),
        in_specs=[
            pl.BlockSpec((1, kk, 1), lambda b, h: (b, 0, 0)),
            pl.BlockSpec((kk, HD), lambda b, h: (b, h)),
            pl.BlockSpec((kk, HD), lambda b, h: (b, h)),
            pl.BlockSpec((kk, HD), lambda b, h: (b, h)),
        ],
        out_specs=pl.BlockSpec((kk, HD), lambda b, h: (b, h)),
        out_shape=jax.ShapeDtypeStruct((BT, D), bf16),
        compiler_params=vlim,
    )(pos_f, q, k, v)

    x = pl.pallas_call(
        _o_kernel,
        grid=(BT // TT,),
        in_specs=[
            pl.BlockSpec((TT, D), lambda t: (t, 0)),
            pl.BlockSpec((D, D), lambda t: (0, 0)),
            pl.BlockSpec((TT, D), lambda t: (t, 0)),
        ],
        out_specs=pl.BlockSpec((TT, D), lambda t: (t, 0)),
        out_shape=jax.ShapeDtypeStruct((BT, D), f32),
        compiler_params=vlim,
    )(ctx, o_wt, sel)

    act = pl.pallas_call(
        _mlp1_kernel,
        grid=(BT // FT,),
        in_specs=[
            pl.BlockSpec((FT, D), lambda t: (t, 0)),
            pl.BlockSpec((D, INTER), lambda t: (0, 0)),
            pl.BlockSpec((D, INTER), lambda t: (0, 0)),
            pl.BlockSpec((D,), lambda t: (0,)),
        ],
        out_specs=pl.BlockSpec((FT, INTER), lambda t: (t, 0)),
        out_shape=jax.ShapeDtypeStruct((BT, INTER), bf16),
        compiler_params=vlim,
    )(x, g_wt, u_wt, ln2_w)

    y = pl.pallas_call(
        _down_kernel,
        grid=(BT // FT,),
        in_specs=[
            pl.BlockSpec((FT, INTER), lambda t: (t, 0)),
            pl.BlockSpec((FT, D), lambda t: (t, 0)),
            pl.BlockSpec((FT, 1), lambda t: (t, 0)),
            pl.BlockSpec((INTER, D), lambda t: (0, 0)),
        ],
        out_specs=pl.BlockSpec((FT, D), lambda t: (t, 0)),
        out_shape=jax.ShapeDtypeStruct((BT, D), f32),
        compiler_params=vlim,
    )(act, x, w_sel, d_wt)

    ST = min(1024, S)
    n_st = S // ST
    bases = jnp.arange(0, S + 1, ST, dtype=jnp.int32)[None, :]
    starts = jnp.sum(idx[:, :, None] < bases[:, None, :], axis=1).astype(
        jnp.int32)

    out = pl.pallas_call(
        _scatter_kernel,
        grid_spec=pltpu.PrefetchScalarGridSpec(
            num_scalar_prefetch=2,
            grid=(B, n_st),
            in_specs=[
                pl.BlockSpec((kk, D), lambda b, t, i_ref, s_ref: (b, 0)),
                pl.BlockSpec((ST, D), lambda b, t, i_ref, s_ref: (b * n_st + t, 0)),
            ],
            out_specs=pl.BlockSpec((ST, D), lambda b, t, i_ref, s_ref: (b * n_st + t, 0)),
        ),
        out_shape=jax.ShapeDtypeStruct((B * S, D), f32),
    )(idx, starts, y, hs2)

    return out.reshape(B, S, D)


# speedup vs baseline: 2.1044x; 1.0066x over previous
import math

import jax
import jax.numpy as jnp
from jax.experimental import pallas as pl
from jax.experimental.pallas import tpu as pltpu

EPS = 1e-6
THETA = 10000.0
NEG = -1e9
HD = 128
f32 = jnp.float32
bf16 = jnp.bfloat16


def _dot1(a, b, dims):
    dn = ((dims), ((), ()))
    return jax.lax.dot_general(a.astype(bf16), b, dn,
                               preferred_element_type=f32)


def _dot1f(a, b, dims):
    return _dot1(a, b.astype(bf16), dims)


def _router_kernel(hs_ref, w_ref, out_ref):
    hb = hs_ref[0].astype(bf16).astype(f32)
    wb = w_ref[...].astype(bf16).astype(f32)
    out_ref[0, 0, :] = jnp.sum(hb * wb, axis=1)


def _gather_kernel(idx_ref, hs_ref, sel_ref):
    b = pl.program_id(0)
    kk = sel_ref.shape[0]

    def body(j, carry):
        r = idx_ref[b, j]
        sel_ref[pl.ds(j, 1), :] = hs_ref[pl.ds(r, 1), :]
        return carry

    jax.lax.fori_loop(0, kk, body, 0)


def _rms(x, w):
    ms = jnp.mean(x * x, axis=1, keepdims=True)
    return x * jax.lax.rsqrt(ms + EPS) * w


def _qkv_kernel(x_ref, qw_ref, kw_ref, vw_ref, qb_ref, kb_ref, vb_ref,
                ln_ref, q_ref, k_ref, v_ref):
    h = _rms(x_ref[...], ln_ref[...])
    hb = h.astype(bf16)
    dn = (((1,), (0,)), ((), ()))
    q_ref[...] = (jax.lax.dot_general(hb, qw_ref[...], dn,
                  preferred_element_type=f32) + qb_ref[...]).astype(bf16)
    k_ref[...] = (jax.lax.dot_general(hb, kw_ref[...], dn,
                  preferred_element_type=f32) + kb_ref[...]).astype(bf16)
    v_ref[...] = (jax.lax.dot_general(hb, vw_ref[...], dn,
                  preferred_element_type=f32) + vb_ref[...]).astype(bf16)


def _attn_kernel(pos_ref, q_ref, k_ref, v_ref, o_ref):
    pos = pos_ref[0]
    half = HD // 2
    i64 = jax.lax.broadcasted_iota(jnp.int32, (1, half), 1).astype(f32)
    inv = jnp.exp(i64 * (-math.log(THETA) / half))
    ang = pos * inv
    c = jnp.cos(ang)
    s = jnp.sin(ang)
    cos = jnp.concatenate([c, c], axis=1)
    sin = jnp.concatenate([s, s], axis=1)

    def rope(x):
        rh = jnp.concatenate([-x[:, half:], x[:, :half]], axis=1)
        return x * cos + rh * sin

    q = rope(q_ref[...].astype(f32))
    k = rope(k_ref[...].astype(f32))
    sc = _dot1f(q, k, ((1,), (1,))) * (1.0 / math.sqrt(HD))
    kk = sc.shape[0]
    ri = jax.lax.broadcasted_iota(jnp.int32, (kk, kk), 0)
    ci = jax.lax.broadcasted_iota(jnp.int32, (kk, kk), 1)
    sc = jnp.where(ri >= ci, sc, NEG)
    m = jnp.max(sc, axis=1, keepdims=True)
    e = jnp.exp(sc - m)
    s = jnp.sum(e, axis=1, keepdims=True)
    o_ref[...] = (_dot1(e, v_ref[...], ((1,), (0,))) / s).astype(bf16)


def _o_kernel(ctx_ref, w_ref, sel_ref, x_ref):
    x_ref[...] = sel_ref[...] + _dot1(ctx_ref[...], w_ref[...], ((1,), (0,)))


def _mlp1_kernel(x_ref, gw_ref, uw_ref, ln_ref, a_ref):
    h = _rms(x_ref[...], ln_ref[...])
    g = _dot1(h, gw_ref[...], ((1,), (0,)))
    u = _dot1(h, uw_ref[...], ((1,), (0,)))
    a_ref[...] = (g * jax.nn.sigmoid(g) * u).astype(bf16)


def _down_kernel(a_ref, x_ref, ws_ref, w_ref, y_ref):
    y = x_ref[...] + _dot1(a_ref[...], w_ref[...], ((1,), (0,)))
    y_ref[...] = y * ws_ref[...]


def _scatter_kernel(idx_ref, st_ref, y_ref, hs_ref, out_ref):
    b = pl.program_id(0)
    t = pl.program_id(1)
    st = out_ref.shape[0]
    out_ref[...] = hs_ref[...]
    lo = st_ref[b, t]
    hi = st_ref[b, t + 1]

    def body(j, carry):
        r = idx_ref[b, j] - t * st
        out_ref[pl.ds(r, 1), :] = y_ref[pl.ds(j, 1), :]
        return carry

    jax.lax.fori_loop(lo, hi, body, 0)


def kernel(hidden_states, position_ids, router_w, router_b, q_w, q_b, k_w,
           k_b, v_w, v_b, o_w, gate_w, up_w, down_w, ln1_w, ln2_w):
    B, S, D = hidden_states.shape
    INTER = gate_w.shape[0]
    H = D // HD
    kk = max(1, S // 2)
    BT = B * kk
    TT = min(256, BT)
    FT = min(256, BT)

    hs2 = hidden_states.reshape(B * S, D)

    q_wt = q_w.T.astype(bf16)
    k_wt = k_w.T.astype(bf16)
    v_wt = v_w.T.astype(bf16)
    o_wt = o_w.T.astype(bf16)
    g_wt = gate_w.T.astype(bf16)
    u_wt = up_w.T.astype(bf16)
    d_wt = down_w.T.astype(bf16)

    vlim = pltpu.CompilerParams(vmem_limit_bytes=63 * 1024 * 1024)

    rw = pl.pallas_call(
        _router_kernel,
        grid=(B,),
        in_specs=[
            pl.BlockSpec((1, S, D), lambda b: (b, 0, 0)),
            pl.BlockSpec((1, D), lambda b: (0, 0)),
        ],
        out_specs=pl.BlockSpec((1, 1, S), lambda b: (b, 0, 0)),
        out_shape=jax.ShapeDtypeStruct((B, 1, S), f32),
    )(hidden_states, router_w)
    rw = rw.reshape(B, S) + router_b[0]

    _, top_idx = jax.lax.top_k(rw, kk)
    idx = jnp.sort(top_idx, axis=1).astype(jnp.int32)
    w_sel = jnp.take_along_axis(rw, idx, axis=1).reshape(BT, 1)
    pos_sel = jnp.take_along_axis(position_ids, idx, axis=1)
    pos_f = pos_sel.astype(f32).reshape(B, kk, 1)

    sel = pl.pallas_call(
        _gather_kernel,
        grid_spec=pltpu.PrefetchScalarGridSpec(
            num_scalar_prefetch=1,
            grid=(B,),
            in_specs=[pl.BlockSpec((S, D), lambda b, i_ref: (b, 0))],
            out_specs=pl.BlockSpec((kk, D), lambda b, i_ref: (b, 0)),
        ),
        out_shape=jax.ShapeDtypeStruct((BT, D), f32),
    )(idx, hs2)

    q, k, v = pl.pallas_call(
        _qkv_kernel,
        grid=(BT // TT,),
        in_specs=[
            pl.BlockSpec((TT, D), lambda t: (t, 0)),
            pl.BlockSpec((D, D), lambda t: (0, 0)),
            pl.BlockSpec((D, D), lambda t: (0, 0)),
            pl.BlockSpec((D, D), lambda t: (0, 0)),
            pl.BlockSpec((D,), lambda t: (0,)),
            pl.BlockSpec((D,), lambda t: (0,)),
            pl.BlockSpec((D,), lambda t: (0,)),
            pl.BlockSpec((D,), lambda t: (0,)),
        ],
        out_specs=[
            pl.BlockSpec((TT, D), lambda t: (t, 0)),
            pl.BlockSpec((TT, D), lambda t: (t, 0)),
            pl.BlockSpec((TT, D), lambda t: (t, 0)),
        ],
        out_shape=[
            jax.ShapeDtypeStruct((BT, D), bf16),
            jax.ShapeDtypeStruct((BT, D), bf16),
            jax.ShapeDtypeStruct((BT, D), bf16),
        ],
        compiler_params=vlim,
    )(sel, q_wt, k_wt, v_wt, q_b, k_b, v_b, ln1_w)

    ctx = pl.pallas_call(
        _attn_kernel,
        grid=(B, H),
        in_specs=[
            pl.BlockSpec((1, kk, 1), lambda b, h: (b, 0, 0)),
            pl.BlockSpec((kk, HD), lambda b, h: (b, h)),
            pl.BlockSpec((kk, HD), lambda b, h: (b, h)),
            pl.BlockSpec((kk, HD), lambda b, h: (b, h)),
        ],
        out_specs=pl.BlockSpec((kk, HD), lambda b, h: (b, h)),
        out_shape=jax.ShapeDtypeStruct((BT, D), bf16),
        compiler_params=vlim,
    )(pos_f, q, k, v)

    x = pl.pallas_call(
        _o_kernel,
        grid=(BT // TT,),
        in_specs=[
            pl.BlockSpec((TT, D), lambda t: (t, 0)),
            pl.BlockSpec((D, D), lambda t: (0, 0)),
            pl.BlockSpec((TT, D), lambda t: (t, 0)),
        ],
        out_specs=pl.BlockSpec((TT, D), lambda t: (t, 0)),
        out_shape=jax.ShapeDtypeStruct((BT, D), f32),
        compiler_params=vlim,
    )(ctx, o_wt, sel)

    act = pl.pallas_call(
        _mlp1_kernel,
        grid=(BT // FT,),
        in_specs=[
            pl.BlockSpec((FT, D), lambda t: (t, 0)),
            pl.BlockSpec((D, INTER), lambda t: (0, 0)),
            pl.BlockSpec((D, INTER), lambda t: (0, 0)),
            pl.BlockSpec((D,), lambda t: (0,)),
        ],
        out_specs=pl.BlockSpec((FT, INTER), lambda t: (t, 0)),
        out_shape=jax.ShapeDtypeStruct((BT, INTER), bf16),
        compiler_params=vlim,
    )(x, g_wt, u_wt, ln2_w)

    y = pl.pallas_call(
        _down_kernel,
        grid=(BT // FT,),
        in_specs=[
            pl.BlockSpec((FT, INTER), lambda t: (t, 0)),
            pl.BlockSpec((FT, D), lambda t: (t, 0)),
            pl.BlockSpec((FT, 1), lambda t: (t, 0)),
            pl.BlockSpec((INTER, D), lambda t: (0, 0)),
        ],
        out_specs=pl.BlockSpec((FT, D), lambda t: (t, 0)),
        out_shape=jax.ShapeDtypeStruct((BT, D), f32),
        compiler_params=vlim,
    )(act, x, w_sel, d_wt)

    ST = min(1024, S)
    n_st = S // ST
    bases = jnp.arange(0, S + 1, ST, dtype=jnp.int32)[None, :]
    starts = jnp.sum(idx[:, :, None] < bases[:, None, :], axis=1).astype(
        jnp.int32)

    out = pl.pallas_call(
        _scatter_kernel,
        grid_spec=pltpu.PrefetchScalarGridSpec(
            num_scalar_prefetch=2,
            grid=(B, n_st),
            in_specs=[
                pl.BlockSpec((kk, D), lambda b, t, i_ref, s_ref: (b, 0)),
                pl.BlockSpec((ST, D), lambda b, t, i_ref, s_ref: (b * n_st + t, 0)),
            ],
            out_specs=pl.BlockSpec((ST, D), lambda b, t, i_ref, s_ref: (b * n_st + t, 0)),
        ),
        out_shape=jax.ShapeDtypeStruct((B * S, D), f32),
    )(idx, starts, y, hs2)

    return out.reshape(B, S, D)
